# Initial kernel scaffold; baseline (speedup 1.0000x reference)
import jax
import jax.numpy as jnp
from jax.experimental import pallas as pl


def kernel(x, edge_index, batch, params):
    raise NotImplementedError("write your pallas kernel here")


# speedup vs baseline: 4.9290x; 4.9290x over previous
import functools

import jax
import jax.numpy as jnp
from jax import lax
from jax.experimental import pallas as pl
from jax.experimental.pallas import tpu as pltpu
from jax.experimental.pallas import tpu_sc as plsc

_SC_CORES = 2
_SC_TILES = 16
_EDGE_CHUNK = 80


def _sc_scatter_partials(x, src, dst):
    N, F = x.shape
    E = src.shape[0]
    n_workers = _SC_CORES * _SC_TILES
    e_per_w = E // n_workers
    assert e_per_w * n_workers == E
    K = _EDGE_CHUNK
    assert e_per_w % K == 0
    n_chunks = e_per_w // K
    rows_per_tile = (N // _SC_TILES + 7) // 8 * 8
    n_pad = rows_per_tile * _SC_TILES

    mesh = plsc.VectorSubcoreMesh(core_axis_name="c", subcore_axis_name="s",
                                  num_cores=_SC_CORES, num_subcores=_SC_TILES)

    @functools.partial(
        pl.kernel,
        mesh=mesh,
        compiler_params=pltpu.CompilerParams(use_tc_tiling_on_sc=False),
        out_type=jax.ShapeDtypeStruct((_SC_CORES, n_pad, F), jnp.float32),
        scratch_types=[
            pltpu.VMEM((K,), jnp.int32),
            pltpu.VMEM((K,), jnp.int32),
            pltpu.VMEM((K, F), jnp.float32),
            pltpu.VMEM_SHARED((n_pad, F), jnp.float32),
            pltpu.SemaphoreType.DMA,
        ],
    )
    def agg_kernel(x_hbm, src_hbm, dst_hbm, zero_hbm, out_hbm,
                   src_v, dst_v, rows_v, acc_sh, sem):
        cid = lax.axis_index("c")
        sid = lax.axis_index("s")
        base = (cid * _SC_TILES + sid) * e_per_w
        pltpu.sync_copy(zero_hbm, acc_sh.at[pl.ds(sid * rows_per_tile, rows_per_tile)])
        plsc.subcore_barrier()

        def body(j, carry):
            off = base + j * K
            pltpu.sync_copy(src_hbm.at[pl.ds(off, K)], src_v)
            pltpu.sync_copy(dst_hbm.at[pl.ds(off, K)], dst_v)
            pltpu.async_copy(x_hbm.at[src_v], rows_v, sem).wait()
            pltpu.sync_copy(rows_v, acc_sh.at[dst_v], add=True)
            return carry

        lax.fori_loop(0, n_chunks, body, 0)
        plsc.subcore_barrier()
        pltpu.sync_copy(acc_sh.at[pl.ds(sid * rows_per_tile, rows_per_tile)],
                        out_hbm.at[cid, pl.ds(sid * rows_per_tile, rows_per_tile)])

    zeros = jnp.zeros((rows_per_tile, F), jnp.float32)
    out = agg_kernel(x, src, dst, zeros)
    return out[:, :N, :]


def _sigmoid(v):
    return 1.0 / (1.0 + jnp.exp(-v))


def _relu(v):
    return jnp.maximum(v, 0.0)


def _elu(v):
    return jnp.where(v > 0, v, jnp.exp(jnp.minimum(v, 0.0)) - 1.0)


def _selu(v):
    alpha = 1.6732632423543772
    scale = 1.0507009873554805
    return scale * jnp.where(v > 0, v, alpha * (jnp.exp(jnp.minimum(v, 0.0)) - 1.0))


def _gelu(v):
    return 0.5 * v * (1.0 + lax.erf(v / jnp.sqrt(2.0).astype(jnp.float32)))


_ACTS = {"sigmoid": _sigmoid, "relu": _relu, "elu": _elu, "selu": _selu,
         "gelu": _gelu, "none": lambda v: v}


def _mlp_rows(inputs, layers, block_rows=1000):
    N = inputs[0].shape[0]
    assert N % block_rows == 0
    grid = N // block_rows
    out_dim = layers[-1][0].shape[1]
    acts = [a for (_, _, a) in layers]

    def body(*refs):
        n_in = len(inputs)
        h = refs[0][...]
        for r in refs[1:n_in]:
            h = h + r[...]
        for li in range(len(layers)):
            wt = refs[n_in + 2 * li][...]
            b = refs[n_in + 2 * li + 1][...]
            h = jnp.dot(h, wt, preferred_element_type=jnp.float32) + b
            h = _ACTS[acts[li]](h)
        refs[-1][...] = h

    in_specs = [pl.BlockSpec((block_rows, a.shape[1]), lambda i: (i, 0))
                for a in inputs]
    operands = list(inputs)
    for (wt, b, _) in layers:
        operands.append(wt)
        operands.append(b.reshape(1, -1))
        in_specs.append(pl.BlockSpec(wt.shape, lambda i: (0, 0)))
        in_specs.append(pl.BlockSpec((1, b.shape[0]), lambda i: (0, 0)))

    return pl.pallas_call(
        body,
        grid=(grid,),
        in_specs=in_specs,
        out_specs=pl.BlockSpec((block_rows, out_dim), lambda i: (i, 0)),
        out_shape=jax.ShapeDtypeStruct((N, out_dim), jnp.float32),
    )(*operands)


def _pool_and_head(x1, x2, x3, batch3d, head_layers, n_graphs, block_rows=1000):
    N, F = x1.shape
    assert N % block_rows == 0
    grid = N // block_rows
    (wa, wb, wc, b_il) = head_layers[0]
    rest = head_layers[1:]

    def kbody(x1_ref, x2_ref, x3_ref, b_ref, wa_ref, wb_ref, wc_ref, bil_ref,
              *refs, _nrest=len(rest)):
        rest_refs = refs[:2 * _nrest]
        o_ref = refs[2 * _nrest]
        p1, p2, p3 = refs[2 * _nrest + 1:2 * _nrest + 4]
        i = pl.program_id(0)

        @pl.when(i == 0)
        def _init():
            p1[...] = jnp.zeros_like(p1)
            p2[...] = jnp.zeros_like(p2)
            p3[...] = jnp.zeros_like(p3)

        seg = b_ref[0, 0, :]
        onehot = (seg[None, :] == lax.broadcasted_iota(
            jnp.int32, (n_graphs, block_rows), 0)).astype(jnp.float32)
        p1[...] += jnp.dot(onehot, x1_ref[...], preferred_element_type=jnp.float32)
        p2[...] += jnp.dot(onehot, x2_ref[...], preferred_element_type=jnp.float32)
        p3[...] += jnp.dot(onehot, x3_ref[...], preferred_element_type=jnp.float32)

        @pl.when(i == grid - 1)
        def _head():
            h = (jnp.dot(p1[...], wa_ref[...], preferred_element_type=jnp.float32)
                 + jnp.dot(p2[...], wb_ref[...], preferred_element_type=jnp.float32)
                 + jnp.dot(p3[...], wc_ref[...], preferred_element_type=jnp.float32)
                 + bil_ref[...])
            h = _sigmoid(h)
            for li in range(_nrest):
                wt = rest_refs[2 * li][...]
                b = rest_refs[2 * li + 1][...]
                h = jnp.dot(h, wt, preferred_element_type=jnp.float32) + b
                h = _ACTS[rest[li][2]](h)
            o_ref[...] = h

    in_specs = [
        pl.BlockSpec((block_rows, F), lambda i: (i, 0)),
        pl.BlockSpec((block_rows, F), lambda i: (i, 0)),
        pl.BlockSpec((block_rows, F), lambda i: (i, 0)),
        pl.BlockSpec((1, 1, block_rows), lambda i: (i, 0, 0)),
        pl.BlockSpec(wa.shape, lambda i: (0, 0)),
        pl.BlockSpec(wb.shape, lambda i: (0, 0)),
        pl.BlockSpec(wc.shape, lambda i: (0, 0)),
        pl.BlockSpec((1, b_il.shape[0]), lambda i: (0, 0)),
    ]
    operands = [x1, x2, x3, batch3d, wa, wb, wc, b_il.reshape(1, -1)]
    for (wt, b, _) in rest:
        operands.append(wt)
        operands.append(b.reshape(1, -1))
        in_specs.append(pl.BlockSpec(wt.shape, lambda i: (0, 0)))
        in_specs.append(pl.BlockSpec((1, b.shape[0]), lambda i: (0, 0)))

    return pl.pallas_call(
        kbody,
        grid=(grid,),
        in_specs=in_specs,
        out_specs=pl.BlockSpec((n_graphs, 1), lambda i: (0, 0)),
        out_shape=jax.ShapeDtypeStruct((n_graphs, 1), jnp.float32),
        scratch_shapes=[
            pltpu.VMEM((n_graphs, F), jnp.float32),
            pltpu.VMEM((n_graphs, F), jnp.float32),
            pltpu.VMEM((n_graphs, F), jnp.float32),
        ],
    )(*operands)


def _lin_t(p):
    return p["W"].T, p["b"]


def _fold_bn(lin, bn):
    inv = bn["gamma"] / jnp.sqrt(bn["var"] + 1e-5)
    w = lin["W"] * inv[:, None]
    b = (lin["b"] - bn["mean"]) * inv + bn["beta"]
    return w.T, b


def _mlp_layers(mlp, bn, post_act):
    wt_il, b_il = _lin_t(mlp["il"])
    wt_1, b_1 = _lin_t(mlp["hl1"])
    wt_2, b_2 = _lin_t(mlp["hl2"])
    wt_3, b_3 = _lin_t(mlp["hl3"])
    wt_ol, b_ol = _fold_bn(mlp["ol"], bn)
    return [
        (wt_il, b_il, "sigmoid"),
        (wt_1, b_1, "relu"),
        (wt_2, b_2, "elu"),
        (wt_3, b_3, "selu"),
        (wt_ol, b_ol, post_act),
    ]


def kernel(x, edge_index, batch, params):
    p = params
    src = edge_index[0].astype(jnp.int32)
    dst = edge_index[1].astype(jnp.int32)
    n_graphs = 128
    N = x.shape[0]
    batch3d = batch.astype(jnp.int32).reshape(N // 1000, 1, 1000)

    w1t, b1 = _lin_t(p["lin1_pre"])
    w2t, b2 = _lin_t(p["lin2_pre"])
    h = _mlp_rows([x], [(w1t, b1, "sigmoid"), (w2t, b2, "relu")])

    a1 = _sc_scatter_partials(h, src, dst)
    x1 = _mlp_rows([h, a1[0], a1[1]], _mlp_layers(p["mlp1"], p["bn1"], "sigmoid"))

    a2 = _sc_scatter_partials(x1, src, dst)
    x2 = _mlp_rows([x1, a2[0], a2[1]], _mlp_layers(p["mlp2"], p["bn2"], "relu"))

    a3 = _sc_scatter_partials(x2, src, dst)
    x3 = _mlp_rows([x2, a3[0], a3[1]], _mlp_layers(p["mlp3"], p["bn3"], "relu"))

    w_il = p["il"]["W"]
    out_sz = x1.shape[1]
    wa = w_il[:, :out_sz].T
    wb = w_il[:, out_sz:2 * out_sz].T
    wc = w_il[:, 2 * out_sz:].T
    head = [(wa, wb, wc, p["il"]["b"])]
    for name, act in (("hl1", "relu"), ("hl2", "elu"), ("hl3", "selu"),
                      ("hl4", "gelu"), ("ol", "none")):
        wt, b = _lin_t(p[name])
        head.append((wt, b, act))

    return _pool_and_head(x1, x2, x3, batch3d, head, n_graphs)


# speedup vs baseline: 5.4959x; 1.1150x over previous
import functools

import numpy as np

import jax
import jax.numpy as jnp
from jax import lax
from jax.experimental import pallas as pl
from jax.experimental.pallas import tpu as pltpu
from jax.experimental.pallas import tpu_sc as plsc

_SC_CORES = 2
_SC_TILES = 16
_EDGE_CHUNK = 125
_NBUF = 2


def _sc_agg(x, src_idx, dst_idx, F_out, feat_split):
    n_rows = x.shape[0] // (2 if feat_split else 1)
    n_chunks, K = src_idx.shape[-2:]
    assert K == _EDGE_CHUNK and n_chunks % _NBUF == 0
    rows_per_tile = (n_rows // _SC_TILES + 7) // 8 * 8
    n_pad = rows_per_tile * _SC_TILES

    mesh = plsc.VectorSubcoreMesh(core_axis_name="c", subcore_axis_name="s",
                                  num_cores=_SC_CORES, num_subcores=_SC_TILES)

    @functools.partial(
        pl.kernel,
        mesh=mesh,
        compiler_params=pltpu.CompilerParams(use_tc_tiling_on_sc=False),
        out_type=jax.ShapeDtypeStruct((_SC_CORES, n_pad, F_out), jnp.float32),
        scratch_types=[
            pltpu.VMEM((n_chunks, K), jnp.int32),
            pltpu.VMEM((n_chunks, K), jnp.int32),
            pltpu.VMEM((_NBUF, K, F_out), jnp.float32),
            pltpu.VMEM_SHARED((n_pad, F_out), jnp.float32),
            pltpu.SemaphoreType.DMA,
            pltpu.SemaphoreType.DMA,
        ],
    )
    def agg_kernel(x_hbm, src_hbm, dst_hbm, zero_hbm, out_hbm,
                   src_v, dst_v, rows_v, acc_sh, sem0, sem1):
        cid = lax.axis_index("c")
        sid = lax.axis_index("s")
        sems = (sem0, sem1)
        if feat_split:
            pltpu.sync_copy(src_hbm.at[cid, sid], src_v)
            pltpu.sync_copy(dst_hbm.at[sid], dst_v)
        else:
            wid = cid * _SC_TILES + sid
            pltpu.sync_copy(src_hbm.at[wid], src_v)
            pltpu.sync_copy(dst_hbm.at[wid], dst_v)
        pltpu.sync_copy(zero_hbm, acc_sh.at[pl.ds(sid * rows_per_tile, rows_per_tile)])
        plsc.subcore_barrier()

        for b in range(_NBUF):
            pltpu.async_copy(x_hbm.at[src_v.at[b]], rows_v.at[b], sems[b])

        def body(g, carry):
            for b in range(_NBUF):
                j = g * _NBUF + b
                pltpu.make_async_copy(x_hbm.at[src_v.at[j]], rows_v.at[b],
                                      sems[b]).wait()
                pltpu.sync_copy(rows_v.at[b], acc_sh.at[dst_v.at[j]], add=True)

                @pl.when(j + _NBUF < n_chunks)
                def _issue_next():
                    pltpu.async_copy(x_hbm.at[src_v.at[j + _NBUF]],
                                     rows_v.at[b], sems[b])
            return carry

        lax.fori_loop(0, n_chunks // _NBUF, body, 0)
        plsc.subcore_barrier()
        pltpu.sync_copy(acc_sh.at[pl.ds(sid * rows_per_tile, rows_per_tile)],
                        out_hbm.at[cid, pl.ds(sid * rows_per_tile, rows_per_tile)])

    zeros = jnp.zeros((rows_per_tile, F_out), jnp.float32)
    return agg_kernel(x, src_idx, dst_idx, zeros)


def _sigmoid(v):
    return 1.0 / (1.0 + jnp.exp(-v))


def _relu(v):
    return jnp.maximum(v, 0.0)


def _expm1(v):
    vp = jnp.where(v > 0, 0.0, v)
    small = vp * (1.0 + vp * (0.5 + vp * (1.0 / 6.0)))
    return jnp.where(vp > -0.01, small, jnp.exp(vp) - 1.0)


def _elu(v):
    return jnp.where(v > 0, v, _expm1(v))


def _selu(v):
    alpha = 1.6732632423543772
    scale = 1.0507009873554805
    return scale * jnp.where(v > 0, v, alpha * _expm1(v))


def _gelu(v):
    sqrt_half = np.sqrt(0.5).astype(np.float32)
    return 0.5 * v * (1.0 + lax.erf(v * sqrt_half))


_ACTS = {"sigmoid": _sigmoid, "relu": _relu, "elu": _elu, "selu": _selu,
         "gelu": _gelu, "none": lambda v: v}


def _mlp_rows(inputs, layers, bn=None, post_act=None, block_rows=1000):
    N = inputs[0].shape[0]
    assert N % block_rows == 0
    grid = N // block_rows
    out_dim = layers[-1][0].shape[1]
    acts = [a for (_, _, a) in layers]
    n_bn = 4 if bn is not None else 0

    def body(*refs):
        n_in = len(inputs)
        h = refs[0][...]
        for r in refs[1:n_in]:
            v = r[...]
            if v.ndim == 3 and v.shape[2] == h.shape[1]:
                h = h + (v[0] + v[1])
            elif v.ndim == 3:
                h = h + jnp.concatenate([v[0], v[1]], axis=-1)
            else:
                h = h + v
        for li in range(len(layers)):
            wt = refs[n_in + 2 * li][...]
            b = refs[n_in + 2 * li + 1][...]
            h = jnp.dot(h, wt, preferred_element_type=jnp.float32) + b
            h = _ACTS[acts[li]](h)
        if bn is not None:
            g_ref, be_ref, m_ref, v_ref = refs[n_in + 2 * len(layers):
                                               n_in + 2 * len(layers) + 4]
            h = (h - m_ref[...]) / jnp.sqrt(v_ref[...] + 1e-5) * g_ref[...] + be_ref[...]
            h = _ACTS[post_act](h)
        refs[-1][...] = h

    in_specs = [
        pl.BlockSpec((2, block_rows, a.shape[2]), lambda i: (0, i, 0))
        if a.ndim == 3 else
        pl.BlockSpec((block_rows, a.shape[1]), lambda i: (i, 0))
        for a in inputs
    ]
    operands = list(inputs)
    for (wt, b, _) in layers:
        operands.append(wt)
        operands.append(b.reshape(1, -1))
        in_specs.append(pl.BlockSpec(wt.shape, lambda i: (0, 0)))
        in_specs.append(pl.BlockSpec((1, b.shape[0]), lambda i: (0, 0)))
    if bn is not None:
        for name in ("gamma", "beta", "mean", "var"):
            operands.append(bn[name].reshape(1, -1))
            in_specs.append(pl.BlockSpec((1, bn[name].shape[0]), lambda i: (0, 0)))

    return pl.pallas_call(
        body,
        grid=(grid,),
        in_specs=in_specs,
        out_specs=pl.BlockSpec((block_rows, out_dim), lambda i: (i, 0)),
        out_shape=jax.ShapeDtypeStruct((N, out_dim), jnp.float32),
    )(*operands)


def _pool_and_head(x1, x2, x3, batch3d, head_layers, n_graphs, block_rows=1000):
    N, F = x1.shape
    assert N % block_rows == 0
    grid = N // block_rows

    def kbody(x1_ref, x2_ref, x3_ref, b_ref, *refs,
              _nl=len(head_layers)):
        w_refs = refs[:2 * _nl]
        o_ref = refs[2 * _nl]
        p1, p2, p3 = refs[2 * _nl + 1:2 * _nl + 4]
        i = pl.program_id(0)

        @pl.when(i == 0)
        def _init():
            p1[...] = jnp.zeros_like(p1)
            p2[...] = jnp.zeros_like(p2)
            p3[...] = jnp.zeros_like(p3)

        seg = b_ref[0, 0, :]
        onehot = (seg[None, :] == lax.broadcasted_iota(
            jnp.int32, (n_graphs, block_rows), 0)).astype(jnp.float32)
        hp = lax.Precision.HIGHEST
        p1[...] += jnp.dot(onehot, x1_ref[...], precision=hp,
                           preferred_element_type=jnp.float32)
        p2[...] += jnp.dot(onehot, x2_ref[...], precision=hp,
                           preferred_element_type=jnp.float32)
        p3[...] += jnp.dot(onehot, x3_ref[...], precision=hp,
                           preferred_element_type=jnp.float32)

        @pl.when(i == grid - 1)
        def _head():
            h = jnp.concatenate([p1[...], p2[...], p3[...]], axis=-1)
            for li in range(_nl):
                wt = w_refs[2 * li][...]
                b = w_refs[2 * li + 1][...]
                h = jnp.dot(h, wt, preferred_element_type=jnp.float32) + b
                h = _ACTS[head_layers[li][2]](h)
            o_ref[...] = h

    in_specs = [
        pl.BlockSpec((block_rows, F), lambda i: (i, 0)),
        pl.BlockSpec((block_rows, F), lambda i: (i, 0)),
        pl.BlockSpec((block_rows, F), lambda i: (i, 0)),
        pl.BlockSpec((1, 1, block_rows), lambda i: (i, 0, 0)),
    ]
    operands = [x1, x2, x3, batch3d]
    for (wt, b, _) in head_layers:
        operands.append(wt)
        operands.append(b.reshape(1, -1))
        in_specs.append(pl.BlockSpec(wt.shape, lambda i: (0, 0)))
        in_specs.append(pl.BlockSpec((1, b.shape[0]), lambda i: (0, 0)))

    return pl.pallas_call(
        kbody,
        grid=(grid,),
        in_specs=in_specs,
        out_specs=pl.BlockSpec((n_graphs, 1), lambda i: (0, 0)),
        out_shape=jax.ShapeDtypeStruct((n_graphs, 1), jnp.float32),
        scratch_shapes=[
            pltpu.VMEM((n_graphs, F), jnp.float32),
            pltpu.VMEM((n_graphs, F), jnp.float32),
            pltpu.VMEM((n_graphs, F), jnp.float32),
        ],
    )(*operands)


def _lin_t(p):
    return p["W"].T, p["b"]


def _mlp_layers(mlp):
    wt_il, b_il = _lin_t(mlp["il"])
    wt_1, b_1 = _lin_t(mlp["hl1"])
    wt_2, b_2 = _lin_t(mlp["hl2"])
    wt_3, b_3 = _lin_t(mlp["hl3"])
    wt_ol, b_ol = _lin_t(mlp["ol"])
    return [
        (wt_il, b_il, "sigmoid"),
        (wt_1, b_1, "relu"),
        (wt_2, b_2, "elu"),
        (wt_3, b_3, "selu"),
        (wt_ol, b_ol, "none"),
    ]


def kernel(x, edge_index, batch, params):
    p = params
    n_workers = _SC_CORES * _SC_TILES
    E = edge_index.shape[1]
    N = x.shape[0]
    D = x.shape[1]
    n_graphs = 128
    K = _EDGE_CHUNK
    src = edge_index[0].astype(jnp.int32)
    dst = edge_index[1].astype(jnp.int32)
    order = jnp.argsort(dst, stable=True)
    src = src[order]
    dst = dst[order]
    src3 = src.reshape(n_workers, E // n_workers // K, K)
    dst3 = dst.reshape(n_workers, E // n_workers // K, K)
    src4 = jnp.stack([src, src + N]).reshape(2, _SC_TILES, E // _SC_TILES // K, K)
    dst16 = dst.reshape(_SC_TILES, E // _SC_TILES // K, K)
    batch3d = batch.astype(jnp.int32).reshape(N // 1000, 1, 1000)

    w1t, b1 = _lin_t(p["lin1_pre"])
    w2t, b2 = _lin_t(p["lin2_pre"])
    h = _mlp_rows([x], [(w1t, b1, "sigmoid"), (w2t, b2, "relu")])

    h_halves = jnp.concatenate([h[:, :D // 2], h[:, D // 2:]], axis=0)
    a1 = _sc_agg(h_halves, src4, dst16, D // 2, feat_split=True)
    x1 = _mlp_rows([h, a1], _mlp_layers(p["mlp1"]), bn=p["bn1"],
                   post_act="sigmoid")

    a2 = _sc_agg(x1, src3, dst3, x1.shape[1], feat_split=False)
    x2 = _mlp_rows([x1, a2], _mlp_layers(p["mlp2"]), bn=p["bn2"],
                   post_act="relu")

    a3 = _sc_agg(x2, src3, dst3, x2.shape[1], feat_split=False)
    x3 = _mlp_rows([x2, a3], _mlp_layers(p["mlp3"]), bn=p["bn3"],
                   post_act="relu")

    head = []
    for name, act in (("il", "sigmoid"), ("hl1", "relu"), ("hl2", "elu"),
                      ("hl3", "selu"), ("hl4", "gelu"), ("ol", "none")):
        wt, b = _lin_t(p[name])
        head.append((wt, b, act))

    return _pool_and_head(x1, x2, x3, batch3d, head, n_graphs)


# speedup vs baseline: 10.7422x; 1.9546x over previous
import functools

import numpy as np

import jax
import jax.numpy as jnp
from jax import lax
from jax.experimental import pallas as pl
from jax.experimental.pallas import tpu as pltpu
from jax.experimental.pallas import tpu_sc as plsc

_SC_CORES = 2
_SC_TILES = 16
_EDGE_CHUNK = 125
_NBUF = 2


def _sc_agg(x, src_idx, dst_idx, F_out, feat_split):
    n_rows = x.shape[0] // (2 if feat_split else 1)
    n_chunks, K = src_idx.shape[-2:]
    assert K == _EDGE_CHUNK and n_chunks % _NBUF == 0
    rows_per_tile = (n_rows // _SC_TILES + 7) // 8 * 8
    n_pad = rows_per_tile * _SC_TILES

    mesh = plsc.VectorSubcoreMesh(core_axis_name="c", subcore_axis_name="s",
                                  num_cores=_SC_CORES, num_subcores=_SC_TILES)

    @functools.partial(
        pl.kernel,
        mesh=mesh,
        compiler_params=pltpu.CompilerParams(use_tc_tiling_on_sc=False),
        out_type=jax.ShapeDtypeStruct((_SC_CORES, n_pad, F_out), jnp.float32),
        scratch_types=[
            pltpu.VMEM((n_chunks, K), jnp.int32),
            pltpu.VMEM((n_chunks, K), jnp.int32),
            pltpu.VMEM((_NBUF, K, F_out), jnp.float32),
            pltpu.VMEM_SHARED((n_pad, F_out), jnp.float32),
            pltpu.SemaphoreType.DMA,
            pltpu.SemaphoreType.DMA,
        ],
    )
    def agg_kernel(x_hbm, src_hbm, dst_hbm, zero_hbm, out_hbm,
                   src_v, dst_v, rows_v, acc_sh, sem0, sem1):
        cid = lax.axis_index("c")
        sid = lax.axis_index("s")
        sems = (sem0, sem1)
        if feat_split:
            pltpu.sync_copy(src_hbm.at[cid, sid], src_v)
            pltpu.sync_copy(dst_hbm.at[sid], dst_v)
        else:
            wid = cid * _SC_TILES + sid
            pltpu.sync_copy(src_hbm.at[wid], src_v)
            pltpu.sync_copy(dst_hbm.at[wid], dst_v)
        pltpu.sync_copy(zero_hbm, acc_sh.at[pl.ds(sid * rows_per_tile, rows_per_tile)])
        plsc.subcore_barrier()

        for b in range(_NBUF):
            pltpu.async_copy(x_hbm.at[src_v.at[b]], rows_v.at[b], sems[b])

        def body(g, carry):
            for b in range(_NBUF):
                j = g * _NBUF + b
                pltpu.make_async_copy(x_hbm.at[src_v.at[j]], rows_v.at[b],
                                      sems[b]).wait()
                pltpu.sync_copy(rows_v.at[b], acc_sh.at[dst_v.at[j]], add=True)

                @pl.when(j + _NBUF < n_chunks)
                def _issue_next():
                    pltpu.async_copy(x_hbm.at[src_v.at[j + _NBUF]],
                                     rows_v.at[b], sems[b])
            return carry

        lax.fori_loop(0, n_chunks // _NBUF, body, 0)
        plsc.subcore_barrier()
        pltpu.sync_copy(acc_sh.at[pl.ds(sid * rows_per_tile, rows_per_tile)],
                        out_hbm.at[cid, pl.ds(sid * rows_per_tile, rows_per_tile)])

    zeros = jnp.zeros((rows_per_tile, F_out), jnp.float32)
    return agg_kernel(x, src_idx, dst_idx, zeros)


def _sigmoid(v):
    return 1.0 / (1.0 + jnp.exp(-v))


def _relu(v):
    return jnp.maximum(v, 0.0)


def _expm1(v):
    vp = jnp.where(v > 0, 0.0, v)
    small = vp * (1.0 + vp * (0.5 + vp * (1.0 / 6.0)))
    return jnp.where(vp > -0.01, small, jnp.exp(vp) - 1.0)


def _elu(v):
    return jnp.where(v > 0, v, _expm1(v))


def _selu(v):
    alpha = 1.6732632423543772
    scale = 1.0507009873554805
    return scale * jnp.where(v > 0, v, alpha * _expm1(v))


def _gelu(v):
    sqrt_half = np.sqrt(0.5).astype(np.float32)
    return 0.5 * v * (1.0 + lax.erf(v * sqrt_half))


_ACTS = {"sigmoid": _sigmoid, "relu": _relu, "elu": _elu, "selu": _selu,
         "gelu": _gelu, "none": lambda v: v}


def _mlp_rows(inputs, layers, bn=None, post_act=None, block_rows=1000):
    N = inputs[0].shape[0]
    assert N % block_rows == 0
    grid = N // block_rows
    out_dim = layers[-1][0].shape[1]
    acts = [a for (_, _, a) in layers]
    n_bn = 4 if bn is not None else 0

    def body(*refs):
        n_in = len(inputs)
        h = refs[0][...]
        for r in refs[1:n_in]:
            v = r[...]
            if v.ndim == 3 and v.shape[2] == h.shape[1]:
                h = h + (v[0] + v[1])
            elif v.ndim == 3:
                h = h + jnp.concatenate([v[0], v[1]], axis=-1)
            else:
                h = h + v
        for li in range(len(layers)):
            wt = refs[n_in + 2 * li][...]
            b = refs[n_in + 2 * li + 1][...]
            h = jnp.dot(h, wt, preferred_element_type=jnp.float32) + b
            h = _ACTS[acts[li]](h)
        if bn is not None:
            g_ref, be_ref, m_ref, v_ref = refs[n_in + 2 * len(layers):
                                               n_in + 2 * len(layers) + 4]
            h = (h - m_ref[...]) / jnp.sqrt(v_ref[...] + 1e-5) * g_ref[...] + be_ref[...]
            h = _ACTS[post_act](h)
        refs[-1][...] = h

    in_specs = [
        pl.BlockSpec((2, block_rows, a.shape[2]), lambda i: (0, i, 0))
        if a.ndim == 3 else
        pl.BlockSpec((block_rows, a.shape[1]), lambda i: (i, 0))
        for a in inputs
    ]
    operands = list(inputs)
    for (wt, b, _) in layers:
        operands.append(wt)
        operands.append(b.reshape(1, -1))
        in_specs.append(pl.BlockSpec(wt.shape, lambda i: (0, 0)))
        in_specs.append(pl.BlockSpec((1, b.shape[0]), lambda i: (0, 0)))
    if bn is not None:
        for name in ("gamma", "beta", "mean", "var"):
            operands.append(bn[name].reshape(1, -1))
            in_specs.append(pl.BlockSpec((1, bn[name].shape[0]), lambda i: (0, 0)))

    return pl.pallas_call(
        body,
        grid=(grid,),
        in_specs=in_specs,
        out_specs=pl.BlockSpec((block_rows, out_dim), lambda i: (i, 0)),
        out_shape=jax.ShapeDtypeStruct((N, out_dim), jnp.float32),
    )(*operands)


def _pool_and_head(x1, x2, x3, batch3d, head_layers, n_graphs, block_rows=1000):
    N, F = x1.shape
    assert N % block_rows == 0
    grid = N // block_rows

    def kbody(x1_ref, x2_ref, x3_ref, b_ref, *refs,
              _nl=len(head_layers)):
        w_refs = refs[:2 * _nl]
        o_ref = refs[2 * _nl]
        p1, p2, p3 = refs[2 * _nl + 1:2 * _nl + 4]
        i = pl.program_id(0)

        @pl.when(i == 0)
        def _init():
            p1[...] = jnp.zeros_like(p1)
            p2[...] = jnp.zeros_like(p2)
            p3[...] = jnp.zeros_like(p3)

        seg = b_ref[0, 0, :]
        onehot = (seg[None, :] == lax.broadcasted_iota(
            jnp.int32, (n_graphs, block_rows), 0)).astype(jnp.float32)
        hp = lax.Precision.HIGHEST
        p1[...] += jnp.dot(onehot, x1_ref[...], precision=hp,
                           preferred_element_type=jnp.float32)
        p2[...] += jnp.dot(onehot, x2_ref[...], precision=hp,
                           preferred_element_type=jnp.float32)
        p3[...] += jnp.dot(onehot, x3_ref[...], precision=hp,
                           preferred_element_type=jnp.float32)

        @pl.when(i == grid - 1)
        def _head():
            h = jnp.concatenate([p1[...], p2[...], p3[...]], axis=-1)
            for li in range(_nl):
                wt = w_refs[2 * li][...]
                b = w_refs[2 * li + 1][...]
                h = jnp.dot(h, wt, preferred_element_type=jnp.float32) + b
                h = _ACTS[head_layers[li][2]](h)
            o_ref[...] = h

    in_specs = [
        pl.BlockSpec((block_rows, F), lambda i: (i, 0)),
        pl.BlockSpec((block_rows, F), lambda i: (i, 0)),
        pl.BlockSpec((block_rows, F), lambda i: (i, 0)),
        pl.BlockSpec((1, 1, block_rows), lambda i: (i, 0, 0)),
    ]
    operands = [x1, x2, x3, batch3d]
    for (wt, b, _) in head_layers:
        operands.append(wt)
        operands.append(b.reshape(1, -1))
        in_specs.append(pl.BlockSpec(wt.shape, lambda i: (0, 0)))
        in_specs.append(pl.BlockSpec((1, b.shape[0]), lambda i: (0, 0)))

    return pl.pallas_call(
        kbody,
        grid=(grid,),
        in_specs=in_specs,
        out_specs=pl.BlockSpec((n_graphs, 1), lambda i: (0, 0)),
        out_shape=jax.ShapeDtypeStruct((n_graphs, 1), jnp.float32),
        scratch_shapes=[
            pltpu.VMEM((n_graphs, F), jnp.float32),
            pltpu.VMEM((n_graphs, F), jnp.float32),
            pltpu.VMEM((n_graphs, F), jnp.float32),
        ],
    )(*operands)


def _lin_t(p):
    return p["W"].T, p["b"]


def _mlp_layers(mlp):
    wt_il, b_il = _lin_t(mlp["il"])
    wt_1, b_1 = _lin_t(mlp["hl1"])
    wt_2, b_2 = _lin_t(mlp["hl2"])
    wt_3, b_3 = _lin_t(mlp["hl3"])
    wt_ol, b_ol = _lin_t(mlp["ol"])
    return [
        (wt_il, b_il, "sigmoid"),
        (wt_1, b_1, "relu"),
        (wt_2, b_2, "elu"),
        (wt_3, b_3, "selu"),
        (wt_ol, b_ol, "none"),
    ]


def kernel(x, edge_index, batch, params):
    p = params
    n_workers = _SC_CORES * _SC_TILES
    E = edge_index.shape[1]
    N = x.shape[0]
    D = x.shape[1]
    n_graphs = 128
    K = _EDGE_CHUNK
    src = edge_index[0].astype(jnp.int32)
    dst = edge_index[1].astype(jnp.int32)
    src3 = src.reshape(n_workers, E // n_workers // K, K)
    dst3 = dst.reshape(n_workers, E // n_workers // K, K)
    src4 = jnp.stack([src, src + N]).reshape(2, _SC_TILES, E // _SC_TILES // K, K)
    dst16 = dst.reshape(_SC_TILES, E // _SC_TILES // K, K)
    batch3d = batch.astype(jnp.int32).reshape(N // 1000, 1, 1000)

    w1t, b1 = _lin_t(p["lin1_pre"])
    w2t, b2 = _lin_t(p["lin2_pre"])
    h = _mlp_rows([x], [(w1t, b1, "sigmoid"), (w2t, b2, "relu")])

    h_halves = jnp.concatenate([h[:, :D // 2], h[:, D // 2:]], axis=0)
    a1 = _sc_agg(h_halves, src4, dst16, D // 2, feat_split=True)
    x1 = _mlp_rows([h, a1], _mlp_layers(p["mlp1"]), bn=p["bn1"],
                   post_act="sigmoid")

    a2 = _sc_agg(x1, src3, dst3, x1.shape[1], feat_split=False)
    x2 = _mlp_rows([x1, a2], _mlp_layers(p["mlp2"]), bn=p["bn2"],
                   post_act="relu")

    a3 = _sc_agg(x2, src3, dst3, x2.shape[1], feat_split=False)
    x3 = _mlp_rows([x2, a3], _mlp_layers(p["mlp3"]), bn=p["bn3"],
                   post_act="relu")

    head = []
    for name, act in (("il", "sigmoid"), ("hl1", "relu"), ("hl2", "elu"),
                      ("hl3", "selu"), ("hl4", "gelu"), ("ol", "none")):
        wt, b = _lin_t(p[name])
        head.append((wt, b, act))

    return _pool_and_head(x1, x2, x3, batch3d, head, n_graphs)
